# baseline (speedup 1.0000x reference)
import functools

import jax
import jax.numpy as jnp
from jax import lax
from jax.experimental import pallas as pl
from jax.experimental.pallas import tpu as pltpu
from jax.experimental.pallas import tpu_sc as plsc

EMB = 32
BATCH = 16384
NROWS = 1000000

NC = 2
NS = 16
NW = NC * NS
BPW = BATCH // NW

PRE = 720896
SUF = NROWS - PRE
PER = PRE // NW
CH = 1024
NCHD = PER // CH

TC_BLK = 16384
TC_GRID = -(-SUF // (2 * TC_BLK))
TC_OFF = PRE // TC_BLK


def _scd_compute(buf, stage, w0, w1):
    def gbody(g, carry):
        acc = jnp.full((16,), 0.0, jnp.float32)
        for j in range(16):
            acc = acc + buf[j, pl.ds(g * 16, 16)] * w0[j]
        for j in range(16):
            acc = acc + buf[16 + j, pl.ds(g * 16, 16)] * w1[j]
        stage[pl.ds(g * 16, 16)] = acc
        return carry

    lax.fori_loop(0, CH // 16, gbody, jnp.int32(0))


def _scd_body(w, ut, it, tu, tm, w_v, buf_u, buf_m, stg_u, stg_m,
              sem_u, sem_m, osem_u, osem_m):
    del osem_m
    wid = lax.axis_index("s") * NC + lax.axis_index("c")
    col0 = wid * PER

    pltpu.sync_copy(w, w_v)
    wu0 = w_v[0, pl.ds(0, 16)]
    wu1 = w_v[0, pl.ds(16, 16)]
    wm0 = w_v[0, pl.ds(32, 16)]
    wm1 = w_v[0, pl.ds(48, 16)]

    nchd = jnp.int32(NCHD)

    def in_fire(tab, buf, sem, c):
        pltpu.async_copy(tab.at[:, pl.ds(col0 + c * CH, CH)], buf, sem)

    def in_wait(tab, buf, sem):
        pltpu.make_async_copy(tab.at[:, pl.ds(col0, CH)], buf, sem).wait()

    in_fire(ut, buf_u, sem_u, jnp.int32(0))

    def cbody(c, carry):
        in_wait(ut, buf_u, sem_u)
        in_fire(it, buf_m, sem_m, c)
        _scd_compute(buf_u, stg_u, wu0, wu1)
        pltpu.sync_copy(stg_u, tu.at[pl.ds(col0 + c * CH, CH)])
        in_wait(it, buf_m, sem_m)
        in_fire(ut, buf_u, sem_u, jnp.minimum(c + 1, nchd - 1))
        _scd_compute(buf_m, stg_m, wm0, wm1)
        pltpu.sync_copy(stg_m, tm.at[pl.ds(col0 + c * CH, CH)])
        return carry

    lax.fori_loop(0, NCHD, cbody, jnp.int32(0))
    in_wait(ut, buf_u, sem_u)


@functools.partial(
    pl.kernel,
    out_type=[
        jax.ShapeDtypeStruct((PRE,), jnp.float32),
        jax.ShapeDtypeStruct((PRE,), jnp.float32),
    ],
    mesh=plsc.VectorSubcoreMesh(core_axis_name="c", subcore_axis_name="s"),
    scratch_types=[
        pltpu.VMEM((1, 2 * EMB), jnp.float32),
        pltpu.VMEM((EMB, CH), jnp.float32),
        pltpu.VMEM((EMB, CH), jnp.float32),
        pltpu.VMEM((CH,), jnp.float32),
        pltpu.VMEM((CH,), jnp.float32),
        pltpu.SemaphoreType.DMA,
        pltpu.SemaphoreType.DMA,
        pltpu.SemaphoreType.DMA,
        pltpu.SemaphoreType.DMA,
    ],
    compiler_params=pltpu.CompilerParams(
        use_tc_tiling_on_sc=True, needs_layout_passes=False
    ),
)
def _sc_dense(w, ut, it, tu, tm, w_v, buf_u, buf_m, stg_u, stg_m,
              sem_u, sem_m, osem_u, osem_m):
    _scd_body(w, ut, it, tu, tm, w_v, buf_u, buf_m, stg_u, stg_m,
              sem_u, sem_m, osem_u, osem_m)


def _tc_body(w_ref, ut0_ref, ut1_ref, it0_ref, it1_ref, tu_ref, tm_ref):
    wu = w_ref[0, 0:EMB].reshape(1, EMB)
    wm = w_ref[0, EMB : 2 * EMB].reshape(1, EMB)
    dn = (((1,), (0,)), ((), ()))
    for src, wv, dst, off in (
        (ut0_ref, wu, tu_ref, 0), (ut1_ref, wu, tu_ref, TC_BLK),
        (it0_ref, wm, tm_ref, 0), (it1_ref, wm, tm_ref, TC_BLK),
    ):
        dst[0, pl.ds(off, TC_BLK)] = lax.dot_general(
            wv, src[...], dn, preferred_element_type=jnp.float32)[0]


@functools.partial(
    pl.pallas_call,
    grid=(TC_GRID,),
    in_specs=[
        pl.BlockSpec((1, 2 * EMB), lambda i: (0, 0)),
        pl.BlockSpec((EMB, TC_BLK), lambda i: (0, TC_OFF + 2 * i)),
        pl.BlockSpec((EMB, TC_BLK), lambda i: (0, TC_OFF + 2 * i + 1)),
        pl.BlockSpec((EMB, TC_BLK), lambda i: (0, TC_OFF + 2 * i)),
        pl.BlockSpec((EMB, TC_BLK), lambda i: (0, TC_OFF + 2 * i + 1)),
    ],
    out_specs=[
        pl.BlockSpec((1, 2 * TC_BLK), lambda i: (0, i)),
        pl.BlockSpec((1, 2 * TC_BLK), lambda i: (0, i)),
    ],
    out_shape=[
        jax.ShapeDtypeStruct((1, SUF), jnp.float32),
        jax.ShapeDtypeStruct((1, SUF), jnp.float32),
    ],
)
def _tc_reduce(w_ref, ut0_ref, ut1_ref, it0_ref, it1_ref, tu_ref, tm_ref):
    _tc_body(w_ref, ut0_ref, ut1_ref, it0_ref, it1_ref, tu_ref, tm_ref)


def _sc_body(tup, tus, tmp_, tms, users, movies, bias, out,
             uidx_v, midx_v, blk_v, row_v, bias_v, out_v, sems):
    wid = lax.axis_index("s") * NC + lax.axis_index("c")
    base = wid * BPW

    pltpu.sync_copy(users.at[pl.ds(base, BPW)], uidx_v)
    pltpu.sync_copy(movies.at[pl.ds(base, BPW)], midx_v)
    pltpu.sync_copy(bias, bias_v)

    cpre = jnp.full((16,), PRE, jnp.int32)
    cpre1 = jnp.full((16,), PRE - 1, jnp.int32)

    def sbody(g, carry):
        o = g * 16
        iu = uidx_v[pl.ds(o, 16)]
        im = midx_v[pl.ds(o, 16)]
        blk_v[0][pl.ds(o, 16)] = lax.shift_right_logical(
            jnp.minimum(iu, cpre1), 3)
        blk_v[1][pl.ds(o, 16)] = lax.shift_right_logical(
            jnp.maximum(iu, cpre) - cpre, 3)
        blk_v[2][pl.ds(o, 16)] = lax.shift_right_logical(
            jnp.minimum(im, cpre1), 3)
        blk_v[3][pl.ds(o, 16)] = lax.shift_right_logical(
            jnp.maximum(im, cpre) - cpre, 3)
        return carry

    lax.fori_loop(0, BPW // 16, sbody, jnp.int32(0))

    cps = []
    for t, (tab, j) in enumerate(((tup, 0), (tus, 1), (tmp_, 2), (tms, 3))):
        cps.append(pltpu.async_copy(tab.at[blk_v[j]], row_v[j], sems[j]))
    for cp in cps:
        cp.wait()

    bvec = bias_v[pl.ds(0, 16)]
    iot = lax.iota(jnp.int32, 16)
    seven = jnp.full((16,), 7, jnp.int32)

    def gbody(g, carry):
        o = g * 16
        rows = o + iot
        iu = uidx_v[pl.ds(o, 16)]
        im = midx_v[pl.ds(o, 16)]
        uoff = jnp.bitwise_and(iu, seven)
        moff = jnp.bitwise_and(im, seven)
        vu_p = plsc.load_gather(row_v[0], [rows, uoff])
        vu_s = plsc.load_gather(row_v[1], [rows, uoff])
        vm_p = plsc.load_gather(row_v[2], [rows, moff])
        vm_s = plsc.load_gather(row_v[3], [rows, moff])
        vu = jnp.where(iu < cpre, vu_p, vu_s)
        vm = jnp.where(im < cpre, vm_p, vm_s)
        out_v[pl.ds(o, 16)] = vu + vm + bvec
        return carry

    lax.fori_loop(0, BPW // 16, gbody, jnp.int32(0))

    pltpu.sync_copy(out_v, out.at[pl.ds(base, BPW)])


@functools.partial(
    pl.kernel,
    out_type=jax.ShapeDtypeStruct((BATCH,), jnp.float32),
    mesh=plsc.VectorSubcoreMesh(core_axis_name="c", subcore_axis_name="s"),
    scratch_types=[
        pltpu.VMEM((BPW,), jnp.int32),
        pltpu.VMEM((BPW,), jnp.int32),
        [pltpu.VMEM((BPW,), jnp.int32) for _ in range(4)],
        [pltpu.VMEM((BPW, 8), jnp.float32) for _ in range(4)],
        pltpu.VMEM((16,), jnp.float32),
        pltpu.VMEM((BPW,), jnp.float32),
        [pltpu.SemaphoreType.DMA for _ in range(4)],
    ],
    compiler_params=pltpu.CompilerParams(
        use_tc_tiling_on_sc=False, needs_layout_passes=False
    ),
)
def _sc_gather(tup, tus, tmp_, tms, users, movies, bias, out,
               uidx_v, midx_v, blk_v, row_v, bias_v, out_v, sems):
    _sc_body(tup, tus, tmp_, tms, users, movies, bias, out,
             uidx_v, midx_v, blk_v, row_v, bias_v, out_v, sems)


def kernel(users, movies, user_table, item_table, W, b):
    users = users.astype(jnp.int32)
    movies = movies.astype(jnp.int32)
    ut_t = user_table.T
    it_t = item_table.T
    tu_pre, tm_pre = _sc_dense(W, ut_t, it_t)
    tu_suf, tm_suf = _tc_reduce(W, ut_t, ut_t, it_t, it_t)
    bias = jnp.full((16,), b[0], jnp.float32)
    out = _sc_gather(
        tu_pre.reshape(PRE // 8, 8),
        tu_suf.reshape(SUF // 8, 8),
        tm_pre.reshape(PRE // 8, 8),
        tm_suf.reshape(SUF // 8, 8),
        users, movies, bias,
    )
    return out.reshape(BATCH, 1)


# speedup vs baseline: 1.1243x; 1.1243x over previous
import functools

import jax
import jax.numpy as jnp
from jax import lax
from jax.experimental import pallas as pl
from jax.experimental.pallas import tpu as pltpu
from jax.experimental.pallas import tpu_sc as plsc

EMB = 32
BATCH = 16384
NROWS = 1000000

NC = 2
NS = 16
NW = NC * NS
BPW = BATCH // NW

TC_BLK = 16384
TC_GRID = -(-NROWS // (2 * TC_BLK))


def _tc_body(w_ref, ut0_ref, ut1_ref, it0_ref, it1_ref, tu_ref, tm_ref):
    wu = w_ref[0, 0:EMB].reshape(1, EMB)
    wm = w_ref[0, EMB : 2 * EMB].reshape(1, EMB)
    dn = (((1,), (0,)), ((), ()))
    for src, wv, dst, off in (
        (ut0_ref, wu, tu_ref, 0), (ut1_ref, wu, tu_ref, TC_BLK),
        (it0_ref, wm, tm_ref, 0), (it1_ref, wm, tm_ref, TC_BLK),
    ):
        dst[0, pl.ds(off, TC_BLK)] = lax.dot_general(
            wv, src[...], dn, preferred_element_type=jnp.float32)[0]


@functools.partial(
    pl.pallas_call,
    grid=(TC_GRID,),
    in_specs=[
        pl.BlockSpec((1, 2 * EMB), lambda i: (0, 0)),
        pl.BlockSpec((EMB, TC_BLK), lambda i: (0, 2 * i)),
        pl.BlockSpec((EMB, TC_BLK), lambda i: (0, 2 * i + 1)),
        pl.BlockSpec((EMB, TC_BLK), lambda i: (0, 2 * i)),
        pl.BlockSpec((EMB, TC_BLK), lambda i: (0, 2 * i + 1)),
    ],
    out_specs=[
        pl.BlockSpec((1, 2 * TC_BLK), lambda i: (0, i)),
        pl.BlockSpec((1, 2 * TC_BLK), lambda i: (0, i)),
    ],
    out_shape=[
        jax.ShapeDtypeStruct((1, NROWS), jnp.float32),
        jax.ShapeDtypeStruct((1, NROWS), jnp.float32),
    ],
)
def _tc_reduce(w_ref, ut0_ref, ut1_ref, it0_ref, it1_ref, tu_ref, tm_ref):
    _tc_body(w_ref, ut0_ref, ut1_ref, it0_ref, it1_ref, tu_ref, tm_ref)


def _sc_body(tu, tm, users, movies, bias, out,
             uidx_v, midx_v, ublk_v, mblk_v, urow_v, mrow_v, bias_v, out_v,
             sem_u, sem_m):
    wid = lax.axis_index("s") * NC + lax.axis_index("c")
    base = wid * BPW

    pltpu.sync_copy(users.at[pl.ds(base, BPW)], uidx_v)
    pltpu.sync_copy(movies.at[pl.ds(base, BPW)], midx_v)
    pltpu.sync_copy(bias, bias_v)

    def sbody(g, carry):
        o = g * 16
        ublk_v[pl.ds(o, 16)] = lax.shift_right_logical(uidx_v[pl.ds(o, 16)], 3)
        mblk_v[pl.ds(o, 16)] = lax.shift_right_logical(midx_v[pl.ds(o, 16)], 3)
        return carry

    lax.fori_loop(0, BPW // 16, sbody, jnp.int32(0))

    cp_u = pltpu.async_copy(tu.at[ublk_v], urow_v, sem_u)
    cp_m = pltpu.async_copy(tm.at[mblk_v], mrow_v, sem_m)
    cp_u.wait()
    cp_m.wait()

    bvec = bias_v[pl.ds(0, 16)]
    iot = lax.iota(jnp.int32, 16)
    seven = jnp.full((16,), 7, jnp.int32)

    def gbody(g, carry):
        o = g * 16
        rows = o + iot
        uoff = jnp.bitwise_and(uidx_v[pl.ds(o, 16)], seven)
        moff = jnp.bitwise_and(midx_v[pl.ds(o, 16)], seven)
        vu = plsc.load_gather(urow_v, [rows, uoff])
        vm = plsc.load_gather(mrow_v, [rows, moff])
        out_v[pl.ds(o, 16)] = vu + vm + bvec
        return carry

    lax.fori_loop(0, BPW // 16, gbody, jnp.int32(0))

    pltpu.sync_copy(out_v, out.at[pl.ds(base, BPW)])


@functools.partial(
    pl.kernel,
    out_type=jax.ShapeDtypeStruct((BATCH,), jnp.float32),
    mesh=plsc.VectorSubcoreMesh(core_axis_name="c", subcore_axis_name="s"),
    scratch_types=[
        pltpu.VMEM((BPW,), jnp.int32),
        pltpu.VMEM((BPW,), jnp.int32),
        pltpu.VMEM((BPW,), jnp.int32),
        pltpu.VMEM((BPW,), jnp.int32),
        pltpu.VMEM((BPW, 8), jnp.float32),
        pltpu.VMEM((BPW, 8), jnp.float32),
        pltpu.VMEM((16,), jnp.float32),
        pltpu.VMEM((BPW,), jnp.float32),
        pltpu.SemaphoreType.DMA,
        pltpu.SemaphoreType.DMA,
    ],
    compiler_params=pltpu.CompilerParams(
        use_tc_tiling_on_sc=False, needs_layout_passes=False
    ),
)
def _sc_gather(tu, tm, users, movies, bias, out,
               uidx_v, midx_v, ublk_v, mblk_v, urow_v, mrow_v, bias_v, out_v,
               sem_u, sem_m):
    _sc_body(tu, tm, users, movies, bias, out,
             uidx_v, midx_v, ublk_v, mblk_v, urow_v, mrow_v, bias_v, out_v,
             sem_u, sem_m)


def kernel(users, movies, user_table, item_table, W, b):
    users = users.astype(jnp.int32)
    movies = movies.astype(jnp.int32)
    ut_t = user_table.T
    it_t = item_table.T
    tu, tm = _tc_reduce(W, ut_t, ut_t, it_t, it_t)
    tu = tu.reshape(NROWS // 8, 8)
    tm = tm.reshape(NROWS // 8, 8)
    bias = jnp.full((16,), b[0], jnp.float32)
    out = _sc_gather(tu, tm, users, movies, bias)
    return out.reshape(BATCH, 1)
